# Initial kernel scaffold; baseline (speedup 1.0000x reference)
import functools

import jax
import jax.numpy as jnp
from jax import lax
from jax.experimental import pallas as pl
from jax.experimental.pallas import tpu as pltpu
from jax.experimental.pallas import tpu_sc as plsc

N_NODES = 10000
N_EDGES = 320000
D = 128

NW = 32
NLOC = 320
NPAD = NW * NLOC
TRASH = NLOC
ACCR = NLOC + 1
LANES = 16
SUBCAP = 1024
CHUNK = 128
ECHUNK = 16000
RPT = D // LANES


def _edge_partition_kernel():
    mesh = plsc.VectorSubcoreMesh(core_axis_name="c", subcore_axis_name="s")

    @functools.partial(
        pl.kernel,
        out_type=(
            jax.ShapeDtypeStruct((NW, LANES * SUBCAP), jnp.int32),
            jax.ShapeDtypeStruct((NW, LANES), jnp.int32),
        ),
        mesh=mesh,
        scratch_types=[
            pltpu.VMEM((LANES * SUBCAP,), jnp.int32),
            pltpu.VMEM((ECHUNK,), jnp.int32),
            pltpu.VMEM((ECHUNK,), jnp.int32),
            pltpu.VMEM((LANES,), jnp.int32),
        ],
    )
    def part(src_hbm, dst_hbm, lists_hbm, counts_hbm, lv, srcb, dstb, cv):
        wid = lax.axis_index("s") * 2 + lax.axis_index("c")
        lo = wid * NLOC
        pad = jnp.full((LANES,), TRASH, jnp.int32)

        def fill(i, _):
            lv[pl.ds(i * LANES, LANES)] = pad
            return 0

        lax.fori_loop(0, SUBCAP, fill, 0)

        lanebase = lax.iota(jnp.int32, LANES) * SUBCAP

        def outer(k, cnt):
            pltpu.sync_copy(src_hbm.at[pl.ds(k * ECHUNK, ECHUNK)], srcb)
            pltpu.sync_copy(dst_hbm.at[pl.ds(k * ECHUNK, ECHUNK)], dstb)

            def inner(j, cnt):
                sv = srcb[pl.ds(j * LANES, LANES)]
                dv = dstb[pl.ds(j * LANES, LANES)]
                dl = dv - lo
                m = (dl >= 0) & (dl < NLOC)
                pk = (sv << 9) | jnp.where(m, dl, 0)
                plsc.store_scatter(lv, [lanebase + cnt], pk, m)
                return cnt + jnp.where(m, 1, 0).astype(jnp.int32)

            return lax.fori_loop(0, ECHUNK // LANES, inner, cnt)

        cnt = lax.fori_loop(0, N_EDGES // ECHUNK, outer,
                            jnp.zeros((LANES,), jnp.int32))
        cv[...] = cnt
        pltpu.sync_copy(lv, lists_hbm.at[wid])
        pltpu.sync_copy(cv, counts_hbm.at[wid])

    return part


def _max_agg_kernel():
    mesh = plsc.VectorSubcoreMesh(core_axis_name="c", subcore_axis_name="s")

    @functools.partial(
        pl.kernel,
        out_type=jax.ShapeDtypeStruct((NPAD, D), jnp.float32),
        mesh=mesh,
        scratch_types=[
            pltpu.VMEM((LANES * SUBCAP,), jnp.int32),
            pltpu.VMEM((LANES,), jnp.int32),
            pltpu.VMEM((ACCR, D), jnp.float32),
            pltpu.VMEM((CHUNK,), jnp.int32),
            pltpu.VMEM((CHUNK,), jnp.int32),
            pltpu.VMEM((CHUNK, D), jnp.float32),
            pltpu.SemaphoreType.DMA,
        ],
    )
    def agg(h_hbm, lists_hbm, counts_hbm, out_hbm, lv, cv, acc, sidx, dloc,
            msg, sem):
        wid = lax.axis_index("s") * 2 + lax.axis_index("c")
        pltpu.sync_copy(lists_hbm.at[wid], lv)
        pltpu.sync_copy(counts_hbm.at[wid], cv)

        ninf = jnp.full((LANES,), -jnp.inf, jnp.float32)
        zero = jnp.zeros((LANES,), jnp.float32)

        def initl(i, _):
            acc[i >> 3, pl.ds((i & 7) * LANES, LANES)] = ninf
            return 0

        lax.fori_loop(0, ACCR * 8, initl, 0)

        def do_sublist(l, _):
            cnt = cv[l]
            nch = (cnt + (CHUNK - 1)) >> 7

            def do_chunk(c, _):
                base = l * SUBCAP + c * CHUNK

                def unpack(j, _):
                    pk = lv[pl.ds(base + j * LANES, LANES)]
                    sidx[pl.ds(j * LANES, LANES)] = pk >> 9
                    dloc[pl.ds(j * LANES, LANES)] = pk & (2 ** 9 - 1)
                    return 0

                lax.fori_loop(0, CHUNK // LANES, unpack, 0)
                pltpu.async_copy(h_hbm.at[sidx], msg, sem).wait()

                def do_edge(e, _):
                    dd = dloc[e]
                    for r in range(RPT):
                        a = acc[dd, pl.ds(r * LANES, LANES)]
                        mv = msg[e, pl.ds(r * LANES, LANES)]
                        acc[dd, pl.ds(r * LANES, LANES)] = jnp.maximum(a, mv)
                    return 0

                lax.fori_loop(0, CHUNK, do_edge, 0)
                return 0

            lax.fori_loop(0, nch, do_chunk, 0)
            return 0

        lax.fori_loop(0, LANES, do_sublist, 0)

        def fixl(i, _):
            r = i >> 3
            cc = (i & 7) * LANES
            v = acc[r, pl.ds(cc, LANES)]
            acc[r, pl.ds(cc, LANES)] = jnp.where(v == ninf, zero, v)
            return 0

        lax.fori_loop(0, NLOC * 8, fixl, 0)
        pltpu.sync_copy(acc.at[pl.ds(0, NLOC)],
                        out_hbm.at[pl.ds(wid * NLOC, NLOC)])

    return agg


_part = _edge_partition_kernel()
_agg = _max_agg_kernel()

_ROWS = NPAD // 4


def _tc_linear(h, w, b, silu_in):
    def body(h_ref, w_ref, b_ref, o_ref):
        hv = h_ref[...]
        if silu_in:
            hv = hv / (1.0 + jnp.exp(-hv))
        o_ref[...] = lax.dot_general(
            hv, w_ref[...], (((1,), (1,)), ((), ())),
            preferred_element_type=jnp.float32,
            precision=lax.Precision.HIGHEST,
        ) + b_ref[...]

    return pl.pallas_call(
        body,
        grid=(NPAD // _ROWS,),
        in_specs=[
            pl.BlockSpec((_ROWS, D), lambda i: (i, 0)),
            pl.BlockSpec((D, D), lambda i: (0, 0)),
            pl.BlockSpec((1, D), lambda i: (0, 0)),
        ],
        out_specs=pl.BlockSpec((_ROWS, D), lambda i: (i, 0)),
        out_shape=jax.ShapeDtypeStruct((NPAD, D), jnp.float32),
    )(h, w, b.reshape(1, D))


def kernel(x, edge_index, W0, b0, W1, b1, W2, b2, W3, b3, W4, b4):
    src = edge_index[0].astype(jnp.int32)
    dst = edge_index[1].astype(jnp.int32)
    xp = jnp.zeros((NPAD, D), jnp.float32).at[:N_NODES].set(x)

    lists, counts = _part(src, dst)

    ws = [W0, W1, W2, W3, W4]
    bs = [b0, b1, b2, b3, b4]
    h = xp
    for i in range(5):
        g = _tc_linear(h, ws[i], bs[i], silu_in=(i > 0))
        h = _agg(g, lists, counts)
    return h[:N_NODES]


# speedup vs baseline: 1.0258x; 1.0258x over previous
import functools

import jax
import jax.numpy as jnp
from jax import lax
from jax.experimental import pallas as pl
from jax.experimental.pallas import tpu as pltpu
from jax.experimental.pallas import tpu_sc as plsc

N_NODES = 10000
N_EDGES = 320000
D = 128

NW = 32
NLOC = 320
NPAD = NW * NLOC
TRASH = NLOC
ACCR = NLOC + 1
LANES = 16
SUBCAP = 1024
CHUNK = 128
ECHUNK = 16000
RPT = D // LANES


def _edge_partition_kernel():
    mesh = plsc.VectorSubcoreMesh(core_axis_name="c", subcore_axis_name="s")

    @functools.partial(
        pl.kernel,
        out_type=(
            jax.ShapeDtypeStruct((NW, LANES * SUBCAP), jnp.int32),
            jax.ShapeDtypeStruct((NW, 128), jnp.int32),
        ),
        mesh=mesh,
        compiler_params=pltpu.CompilerParams(needs_layout_passes=False),
        scratch_types=[
            pltpu.VMEM((LANES * SUBCAP,), jnp.int32),
            pltpu.VMEM((ECHUNK,), jnp.int32),
            pltpu.VMEM((ECHUNK,), jnp.int32),
            pltpu.VMEM((128,), jnp.int32),
        ],
    )
    def part(src_hbm, dst_hbm, lists_hbm, counts_hbm, lv, srcb, dstb, cv):
        wid = lax.axis_index("s") * 2 + lax.axis_index("c")
        lo = wid * NLOC
        pad = jnp.full((LANES,), TRASH, jnp.int32)

        def fill(i, _):
            lv[pl.ds(i * LANES, LANES)] = pad
            return 0

        lax.fori_loop(0, SUBCAP, fill, 0)

        lanebase = lax.iota(jnp.int32, LANES) * SUBCAP

        def outer(k, cnt):
            pltpu.sync_copy(src_hbm.at[pl.ds(k * ECHUNK, ECHUNK)], srcb)
            pltpu.sync_copy(dst_hbm.at[pl.ds(k * ECHUNK, ECHUNK)], dstb)

            def inner(j, cnt):
                sv = srcb[pl.ds(j * LANES, LANES)]
                dv = dstb[pl.ds(j * LANES, LANES)]
                dl = dv - lo
                m = (dl >= 0) & (dl < NLOC)
                pk = (sv << 9) | jnp.where(m, dl, 0)
                plsc.store_scatter(lv, [lanebase + cnt], pk, mask=m)
                return cnt + jnp.where(m, 1, 0).astype(jnp.int32)

            return lax.fori_loop(0, ECHUNK // LANES, inner, cnt)

        cnt = lax.fori_loop(0, N_EDGES // ECHUNK, outer,
                            jnp.zeros((LANES,), jnp.int32))
        cv[pl.ds(0, LANES)] = cnt
        pltpu.sync_copy(lv, lists_hbm.at[wid])
        pltpu.sync_copy(cv, counts_hbm.at[wid])

    return part


def _max_agg_kernel():
    mesh = plsc.VectorSubcoreMesh(core_axis_name="c", subcore_axis_name="s")

    @functools.partial(
        pl.kernel,
        out_type=jax.ShapeDtypeStruct((NPAD * D,), jnp.float32),
        mesh=mesh,
        compiler_params=pltpu.CompilerParams(needs_layout_passes=False),
        scratch_types=[
            pltpu.VMEM((LANES * SUBCAP,), jnp.int32),
            pltpu.VMEM((128,), jnp.int32),
            pltpu.VMEM((ACCR * D,), jnp.float32),
            pltpu.VMEM((CHUNK,), jnp.int32),
            pltpu.VMEM((CHUNK,), jnp.int32),
            pltpu.VMEM((CHUNK, D), jnp.float32),
            pltpu.SemaphoreType.DMA,
        ],
    )
    def agg(h_hbm, lists_hbm, counts_hbm, out_hbm, lv, cv, acc, sidx, dloc,
            msg, sem):
        wid = lax.axis_index("s") * 2 + lax.axis_index("c")
        pltpu.sync_copy(lists_hbm.at[wid], lv)
        pltpu.sync_copy(counts_hbm.at[wid], cv)

        ninf = jnp.full((LANES,), -jnp.inf, jnp.float32)
        zero = jnp.zeros((LANES,), jnp.float32)

        def initl(i, _):
            acc[pl.ds(i * LANES, LANES)] = ninf
            return 0

        lax.fori_loop(0, ACCR * (D // LANES), initl, 0)

        def do_sublist(l, _):
            cnt = cv[pl.ds(l, LANES)][0]
            nch = (cnt + (CHUNK - 1)) >> 7

            def do_chunk(c, _):
                base = l * SUBCAP + c * CHUNK

                def unpack(j, _):
                    pk = lv[pl.ds(base + j * LANES, LANES)]
                    sidx[pl.ds(j * LANES, LANES)] = pk >> 9
                    dloc[pl.ds(j * LANES, LANES)] = pk & (2 ** 9 - 1)
                    return 0

                lax.fori_loop(0, CHUNK // LANES, unpack, 0)
                pltpu.async_copy(h_hbm.at[sidx], msg, sem).wait()

                def do_group(g, _):
                    dv = dloc[pl.ds(g * LANES, LANES)]
                    for lane in range(LANES):
                        off = dv[lane] * D
                        erow = g * LANES + lane
                        for r in range(RPT):
                            a = acc[pl.ds(off + r * LANES, LANES)]
                            mv = msg[erow, pl.ds(r * LANES, LANES)]
                            acc[pl.ds(off + r * LANES, LANES)] = (
                                jnp.maximum(a, mv))
                    return 0

                lax.fori_loop(0, CHUNK // LANES, do_group, 0)
                return 0

            lax.fori_loop(0, nch, do_chunk, 0)
            return 0

        lax.fori_loop(0, LANES, do_sublist, 0)

        def fixl(i, _):
            v = acc[pl.ds(i * LANES, LANES)]
            acc[pl.ds(i * LANES, LANES)] = jnp.where(v == ninf, zero, v)
            return 0

        lax.fori_loop(0, NLOC * (D // LANES), fixl, 0)
        pltpu.sync_copy(acc.at[pl.ds(0, NLOC * D)],
                        out_hbm.at[pl.ds(wid * NLOC * D, NLOC * D)])

    return agg


_part = _edge_partition_kernel()
_agg = _max_agg_kernel()

_ROWS = NPAD // 4


def _tc_linear(h, w, b, silu_in):
    def body(h_ref, w_ref, b_ref, o_ref):
        hv = h_ref[...]
        if silu_in:
            hv = hv / (1.0 + jnp.exp(-hv))
        o_ref[...] = lax.dot_general(
            hv, w_ref[...], (((1,), (1,)), ((), ())),
            preferred_element_type=jnp.float32,
            precision=lax.Precision.HIGHEST,
        ) + b_ref[...]

    return pl.pallas_call(
        body,
        grid=(NPAD // _ROWS,),
        in_specs=[
            pl.BlockSpec((_ROWS, D), lambda i: (i, 0)),
            pl.BlockSpec((D, D), lambda i: (0, 0)),
            pl.BlockSpec((1, D), lambda i: (0, 0)),
        ],
        out_specs=pl.BlockSpec((_ROWS, D), lambda i: (i, 0)),
        out_shape=jax.ShapeDtypeStruct((NPAD, D), jnp.float32),
    )(h, w, b.reshape(1, D))


def kernel(x, edge_index, W0, b0, W1, b1, W2, b2, W3, b3, W4, b4):
    src = edge_index[0].astype(jnp.int32)
    dst = edge_index[1].astype(jnp.int32)
    xp = jnp.zeros((NPAD, D), jnp.float32).at[:N_NODES].set(x)

    lists, counts = _part(src, dst)

    ws = [W0, W1, W2, W3, W4]
    bs = [b0, b1, b2, b3, b4]
    h = xp
    for i in range(5):
        g = _tc_linear(h, ws[i], bs[i], silu_in=(i > 0))
        h = _agg(g, lists, counts).reshape(NPAD, D)
    return h[:N_NODES]


# speedup vs baseline: 2.6442x; 2.5777x over previous
import functools

import jax
import jax.numpy as jnp
from jax import lax
from jax.experimental import pallas as pl
from jax.experimental.pallas import tpu as pltpu
from jax.experimental.pallas import tpu_sc as plsc

N_NODES = 10000
N_EDGES = 320000
D = 128

NW = 32
NLOC = 320
NPAD = NW * NLOC
TRASH = NLOC
ACCR = NLOC + 1
LANES = 16
SUBCAP = 1024
CAP = LANES * SUBCAP
CHUNK = 256
ECHUNK = 16000
RPT = D // LANES


def _edge_partition_kernel():
    mesh = plsc.VectorSubcoreMesh(core_axis_name="c", subcore_axis_name="s")

    @functools.partial(
        pl.kernel,
        out_type=(
            jax.ShapeDtypeStruct((NW, CAP), jnp.int32),
            jax.ShapeDtypeStruct((NW, 128), jnp.int32),
        ),
        mesh=mesh,
        compiler_params=pltpu.CompilerParams(needs_layout_passes=False),
        scratch_types=[
            pltpu.VMEM((CAP,), jnp.int32),
            pltpu.VMEM((CAP,), jnp.int32),
            pltpu.VMEM((ECHUNK,), jnp.int32),
            pltpu.VMEM((ECHUNK,), jnp.int32),
            pltpu.VMEM((128,), jnp.int32),
        ],
    )
    def part(src_hbm, dst_hbm, lists_hbm, counts_hbm, lv, lvc, srcb, dstb,
             cv):
        wid = lax.axis_index("s") * 2 + lax.axis_index("c")
        lo = wid * NLOC
        pad = jnp.full((LANES,), TRASH, jnp.int32)

        def fill(i, _):
            lv[pl.ds(i * LANES, LANES)] = pad
            return 0

        lax.fori_loop(0, SUBCAP, fill, 0)

        lanebase = lax.iota(jnp.int32, LANES) * SUBCAP

        def outer(k, cnt):
            pltpu.sync_copy(src_hbm.at[pl.ds(k * ECHUNK, ECHUNK)], srcb)
            pltpu.sync_copy(dst_hbm.at[pl.ds(k * ECHUNK, ECHUNK)], dstb)

            def inner(j, cnt):
                sv = srcb[pl.ds(j * LANES, LANES)]
                dv = dstb[pl.ds(j * LANES, LANES)]
                dl = dv - lo
                m = (dl >= 0) & (dl < NLOC)
                pk = (sv << 9) | jnp.where(m, dl, 0)
                plsc.store_scatter(lv, [lanebase + cnt], pk, mask=m)
                return cnt + jnp.where(m, 1, 0).astype(jnp.int32)

            return lax.fori_loop(0, ECHUNK // LANES, inner, cnt)

        cnt = lax.fori_loop(0, N_EDGES // ECHUNK, outer,
                            jnp.zeros((LANES,), jnp.int32))

        base = jnp.int32(0)
        for l in range(LANES):
            nv = (cnt[l] + (LANES - 1)) >> 4
            src_base = l * SUBCAP

            def cp(j, _, b=base, s=src_base):
                lvc[pl.ds(b + j * LANES, LANES)] = lv[pl.ds(s + j * LANES,
                                                            LANES)]
                return 0

            lax.fori_loop(0, nv, cp, 0)
            base = base + nv * LANES
        for j in range(CHUNK // LANES):
            lvc[pl.ds(base + j * LANES, LANES)] = pad

        cv[pl.ds(0, LANES)] = jnp.full((LANES,), 1, jnp.int32) * base
        pltpu.sync_copy(lvc, lists_hbm.at[wid])
        pltpu.sync_copy(cv, counts_hbm.at[wid])

    return part


def _max_agg_kernel():
    mesh = plsc.VectorSubcoreMesh(core_axis_name="c", subcore_axis_name="s")

    @functools.partial(
        pl.kernel,
        out_type=jax.ShapeDtypeStruct((NPAD * D,), jnp.float32),
        mesh=mesh,
        compiler_params=pltpu.CompilerParams(needs_layout_passes=False),
        scratch_types=[
            pltpu.VMEM((CAP,), jnp.int32),
            pltpu.VMEM((128,), jnp.int32),
            pltpu.VMEM((ACCR * D,), jnp.float32),
            pltpu.VMEM((CHUNK,), jnp.int32),
            pltpu.VMEM((CHUNK,), jnp.int32),
            pltpu.VMEM((CHUNK, D), jnp.float32),
            pltpu.SemaphoreType.DMA,
            pltpu.VMEM((CHUNK,), jnp.int32),
            pltpu.VMEM((CHUNK,), jnp.int32),
            pltpu.VMEM((CHUNK, D), jnp.float32),
            pltpu.SemaphoreType.DMA,
        ],
    )
    def agg(h_hbm, lists_hbm, counts_hbm, out_hbm, lv, cv, acc,
            sidx0, dloc0, msg0, sem0, sidx1, dloc1, msg1, sem1):
        wid = lax.axis_index("s") * 2 + lax.axis_index("c")
        pltpu.sync_copy(lists_hbm.at[wid], lv)
        pltpu.sync_copy(counts_hbm.at[wid], cv)
        total = cv[pl.ds(0, LANES)][0]
        nch = (total + (CHUNK - 1)) >> 8

        ninf = jnp.full((LANES,), -jnp.inf, jnp.float32)
        zero = jnp.zeros((LANES,), jnp.float32)

        def initrow(rr, _):
            for k in range(RPT):
                acc[pl.ds(rr * D + k * LANES, LANES)] = ninf
            return 0

        lax.fori_loop(0, ACCR, initrow, 0)

        bufs = ((sidx0, dloc0, msg0, sem0), (sidx1, dloc1, msg1, sem1))

        def issue(b, c):
            sidx, dloc, msg, sem = bufs[b]
            base = c * CHUNK

            def unpack(j, _):
                pk = lv[pl.ds(base + j * LANES, LANES)]
                sidx[pl.ds(j * LANES, LANES)] = pk >> 9
                dloc[pl.ds(j * LANES, LANES)] = pk & (2 ** 9 - 1)
                return 0

            lax.fori_loop(0, CHUNK // LANES, unpack, 0)
            pltpu.async_copy(h_hbm.at[sidx], msg, sem)

        @pl.when(nch > 0)
        def _():
            issue(0, 0)

        @pl.when(nch > 1)
        def _():
            issue(1, 1)

        def louter(c2, _):
            for b in range(2):
                c = c2 * 2 + b
                sidx, dloc, msg, sem = bufs[b]

                @pl.when(c < nch)
                def _(c=c, sidx=sidx, dloc=dloc, msg=msg, sem=sem, b=b):
                    pltpu.make_async_copy(h_hbm.at[sidx], msg, sem).wait()

                    def do_group(g, _):
                        dv = dloc[pl.ds(g * LANES, LANES)]
                        for lane in range(LANES):
                            off = dv[lane] * D
                            erow = g * LANES + lane
                            for r in range(RPT):
                                a = acc[pl.ds(off + r * LANES, LANES)]
                                mv = msg[erow, pl.ds(r * LANES, LANES)]
                                acc[pl.ds(off + r * LANES, LANES)] = (
                                    jnp.maximum(a, mv))
                        return 0

                    lax.fori_loop(0, CHUNK // LANES, do_group, 0)

                    @pl.when(c + 2 < nch)
                    def _():
                        issue(b, c + 2)

            return 0

        lax.fori_loop(0, (nch + 1) >> 1, louter, 0)

        def fixrow(rr, _):
            for k in range(RPT):
                v = acc[pl.ds(rr * D + k * LANES, LANES)]
                acc[pl.ds(rr * D + k * LANES, LANES)] = (
                    jnp.where(v == ninf, zero, v))
            return 0

        lax.fori_loop(0, NLOC, fixrow, 0)
        pltpu.sync_copy(acc.at[pl.ds(0, NLOC * D)],
                        out_hbm.at[pl.ds(wid * NLOC * D, NLOC * D)])

    return agg


_part = _edge_partition_kernel()
_agg = _max_agg_kernel()

_ROWS = NPAD // 4


def _tc_linear(h, w, b, silu_in):
    def body(h_ref, w_ref, b_ref, o_ref):
        hv = h_ref[...]
        if silu_in:
            hv = hv / (1.0 + jnp.exp(-hv))
        o_ref[...] = lax.dot_general(
            hv, w_ref[...], (((1,), (1,)), ((), ())),
            preferred_element_type=jnp.float32,
            precision=lax.Precision.HIGHEST,
        ) + b_ref[...]

    return pl.pallas_call(
        body,
        grid=(NPAD // _ROWS,),
        in_specs=[
            pl.BlockSpec((_ROWS, D), lambda i: (i, 0)),
            pl.BlockSpec((D, D), lambda i: (0, 0)),
            pl.BlockSpec((1, D), lambda i: (0, 0)),
        ],
        out_specs=pl.BlockSpec((_ROWS, D), lambda i: (i, 0)),
        out_shape=jax.ShapeDtypeStruct((NPAD, D), jnp.float32),
    )(h, w, b.reshape(1, D))


def kernel(x, edge_index, W0, b0, W1, b1, W2, b2, W3, b3, W4, b4):
    src = edge_index[0].astype(jnp.int32)
    dst = edge_index[1].astype(jnp.int32)
    xp = jnp.zeros((NPAD, D), jnp.float32).at[:N_NODES].set(x)

    lists, counts = _part(src, dst)

    ws = [W0, W1, W2, W3, W4]
    bs = [b0, b1, b2, b3, b4]
    h = xp
    for i in range(5):
        g = _tc_linear(h, ws[i], bs[i], silu_in=(i > 0))
        h = _agg(g, lists, counts).reshape(NPAD, D)
    return h[:N_NODES]


# speedup vs baseline: 3.0731x; 1.1622x over previous
import functools

import jax
import jax.numpy as jnp
from jax import lax
from jax.experimental import pallas as pl
from jax.experimental.pallas import tpu as pltpu
from jax.experimental.pallas import tpu_sc as plsc

N_NODES = 10000
N_EDGES = 320000
D = 128

NW = 32
NLOC = 320
NPAD = NW * NLOC
TRASH = NLOC
ACCR = NLOC + 1
LANES = 16
SUBCAP = 1024
CAP = LANES * SUBCAP
CHUNK = 128
NBUF = 4
ECHUNK = 16000
RPT = D // LANES


def _edge_partition_kernel():
    mesh = plsc.VectorSubcoreMesh(core_axis_name="c", subcore_axis_name="s")

    @functools.partial(
        pl.kernel,
        out_type=(
            jax.ShapeDtypeStruct((NW, CAP), jnp.int32),
            jax.ShapeDtypeStruct((NW, 128), jnp.int32),
        ),
        mesh=mesh,
        compiler_params=pltpu.CompilerParams(needs_layout_passes=False),
        scratch_types=[
            pltpu.VMEM((CAP,), jnp.int32),
            pltpu.VMEM((CAP,), jnp.int32),
            pltpu.VMEM((ECHUNK,), jnp.int32),
            pltpu.VMEM((ECHUNK,), jnp.int32),
            pltpu.VMEM((128,), jnp.int32),
        ],
    )
    def part(src_hbm, dst_hbm, lists_hbm, counts_hbm, lv, lvc, srcb, dstb,
             cv):
        wid = lax.axis_index("s") * 2 + lax.axis_index("c")
        lo = wid * NLOC
        pad = jnp.full((LANES,), TRASH, jnp.int32)

        def fill(i, _):
            lv[pl.ds(i * LANES, LANES)] = pad
            return 0

        lax.fori_loop(0, SUBCAP, fill, 0)

        lanebase = lax.iota(jnp.int32, LANES) * SUBCAP

        def outer(k, cnt):
            pltpu.sync_copy(src_hbm.at[pl.ds(k * ECHUNK, ECHUNK)], srcb)
            pltpu.sync_copy(dst_hbm.at[pl.ds(k * ECHUNK, ECHUNK)], dstb)

            def inner(j, cnt):
                sv = srcb[pl.ds(j * LANES, LANES)]
                dv = dstb[pl.ds(j * LANES, LANES)]
                dl = dv - lo
                m = (dl >= 0) & (dl < NLOC)
                pk = (sv << 9) | jnp.where(m, dl, 0)
                plsc.store_scatter(lv, [lanebase + cnt], pk, mask=m)
                return cnt + jnp.where(m, 1, 0).astype(jnp.int32)

            return lax.fori_loop(0, ECHUNK // LANES, inner, cnt)

        cnt = lax.fori_loop(0, N_EDGES // ECHUNK, outer,
                            jnp.zeros((LANES,), jnp.int32))

        base = jnp.int32(0)
        for l in range(LANES):
            nv = (cnt[l] + (LANES - 1)) >> 4
            src_base = l * SUBCAP

            def cp(j, _, b=base, s=src_base):
                lvc[pl.ds(b + j * LANES, LANES)] = lv[pl.ds(s + j * LANES,
                                                            LANES)]
                return 0

            lax.fori_loop(0, nv, cp, 0)
            base = base + nv * LANES
        for j in range(CHUNK // LANES):
            lvc[pl.ds(base + j * LANES, LANES)] = pad

        cv[pl.ds(0, LANES)] = jnp.full((LANES,), 1, jnp.int32) * base
        pltpu.sync_copy(lvc, lists_hbm.at[wid])
        pltpu.sync_copy(cv, counts_hbm.at[wid])

    return part


def _max_agg_kernel():
    mesh = plsc.VectorSubcoreMesh(core_axis_name="c", subcore_axis_name="s")

    @functools.partial(
        pl.kernel,
        out_type=jax.ShapeDtypeStruct((NPAD * D,), jnp.float32),
        mesh=mesh,
        compiler_params=pltpu.CompilerParams(needs_layout_passes=False),
        scratch_types=[
            pltpu.VMEM((CAP,), jnp.int32),
            pltpu.VMEM((128,), jnp.int32),
            pltpu.VMEM((ACCR * D,), jnp.float32),
        ] + [
            t for _ in range(NBUF) for t in (
                pltpu.VMEM((CHUNK,), jnp.int32),
                pltpu.VMEM((CHUNK,), jnp.int32),
                pltpu.VMEM((CHUNK, D), jnp.float32),
                pltpu.SemaphoreType.DMA,
            )
        ],
    )
    def agg(h_hbm, lists_hbm, counts_hbm, out_hbm, lv, cv, acc, *bufargs):
        wid = lax.axis_index("s") * 2 + lax.axis_index("c")
        pltpu.sync_copy(lists_hbm.at[wid], lv)
        pltpu.sync_copy(counts_hbm.at[wid], cv)
        total = cv[pl.ds(0, LANES)][0]
        nch = (total + (CHUNK - 1)) // CHUNK

        ninf = jnp.full((LANES,), -jnp.inf, jnp.float32)
        zero = jnp.zeros((LANES,), jnp.float32)

        def initrow(rr, _):
            for k in range(RPT):
                acc[pl.ds(rr * D + k * LANES, LANES)] = ninf
            return 0

        lax.fori_loop(0, ACCR, initrow, 0)

        bufs = tuple(tuple(bufargs[4 * b:4 * b + 4]) for b in range(NBUF))

        def issue(b, c):
            sidx, dloc, msg, sem = bufs[b]
            base = c * CHUNK

            def unpack(j, _):
                pk = lv[pl.ds(base + j * LANES, LANES)]
                sidx[pl.ds(j * LANES, LANES)] = pk >> 9
                dloc[pl.ds(j * LANES, LANES)] = pk & (2 ** 9 - 1)
                return 0

            lax.fori_loop(0, CHUNK // LANES, unpack, 0)
            pltpu.async_copy(h_hbm.at[sidx], msg, sem)

        for b in range(NBUF):
            @pl.when(nch > b)
            def _(b=b):
                issue(b, b)

        def louter(c2, _):
            for b in range(NBUF):
                c = c2 * NBUF + b
                sidx, dloc, msg, sem = bufs[b]

                @pl.when(c < nch)
                def _(c=c, sidx=sidx, dloc=dloc, msg=msg, sem=sem, b=b):
                    pltpu.make_async_copy(h_hbm.at[sidx], msg, sem).wait()

                    def do_group(g, _):
                        dv = dloc[pl.ds(g * LANES, LANES)]
                        for lane in range(LANES):
                            off = dv[lane] * D
                            erow = g * LANES + lane
                            for r in range(RPT):
                                a = acc[pl.ds(off + r * LANES, LANES)]
                                mv = msg[erow, pl.ds(r * LANES, LANES)]
                                acc[pl.ds(off + r * LANES, LANES)] = (
                                    jnp.maximum(a, mv))
                        return 0

                    lax.fori_loop(0, CHUNK // LANES, do_group, 0)

                    @pl.when(c + NBUF < nch)
                    def _():
                        issue(b, c + NBUF)

            return 0

        lax.fori_loop(0, (nch + NBUF - 1) // NBUF, louter, 0)

        def fixrow(rr, _):
            for k in range(RPT):
                v = acc[pl.ds(rr * D + k * LANES, LANES)]
                acc[pl.ds(rr * D + k * LANES, LANES)] = (
                    jnp.where(v == ninf, zero, v))
            return 0

        lax.fori_loop(0, NLOC, fixrow, 0)
        pltpu.sync_copy(acc.at[pl.ds(0, NLOC * D)],
                        out_hbm.at[pl.ds(wid * NLOC * D, NLOC * D)])

    return agg


_part = _edge_partition_kernel()
_agg = _max_agg_kernel()

_ROWS = NPAD // 4


def _tc_linear(h, w, b, silu_in):
    def body(h_ref, w_ref, b_ref, o_ref):
        hv = h_ref[...]
        if silu_in:
            hv = hv / (1.0 + jnp.exp(-hv))
        o_ref[...] = lax.dot_general(
            hv, w_ref[...], (((1,), (1,)), ((), ())),
            preferred_element_type=jnp.float32,
            precision=lax.Precision.HIGHEST,
        ) + b_ref[...]

    return pl.pallas_call(
        body,
        grid=(NPAD // _ROWS,),
        in_specs=[
            pl.BlockSpec((_ROWS, D), lambda i: (i, 0)),
            pl.BlockSpec((D, D), lambda i: (0, 0)),
            pl.BlockSpec((1, D), lambda i: (0, 0)),
        ],
        out_specs=pl.BlockSpec((_ROWS, D), lambda i: (i, 0)),
        out_shape=jax.ShapeDtypeStruct((NPAD, D), jnp.float32),
    )(h, w, b.reshape(1, D))


def kernel(x, edge_index, W0, b0, W1, b1, W2, b2, W3, b3, W4, b4):
    src = edge_index[0].astype(jnp.int32)
    dst = edge_index[1].astype(jnp.int32)
    xp = jnp.zeros((NPAD, D), jnp.float32).at[:N_NODES].set(x)

    lists, counts = _part(src, dst)

    ws = [W0, W1, W2, W3, W4]
    bs = [b0, b1, b2, b3, b4]
    h = xp
    for i in range(5):
        g = _tc_linear(h, ws[i], bs[i], silu_in=(i > 0))
        h = _agg(g, lists, counts).reshape(NPAD, D)
    return h[:N_NODES]


# speedup vs baseline: 4.5211x; 1.4712x over previous
import functools

import jax
import jax.numpy as jnp
from jax import lax
from jax.experimental import pallas as pl
from jax.experimental.pallas import tpu as pltpu
from jax.experimental.pallas import tpu_sc as plsc

N_NODES = 10000
N_EDGES = 320000
D = 128

NW = 32
NLOC = 320
NPAD = NW * NLOC
TRASH = NLOC
ACCR = NLOC + 1
LANES = 16
SUBCAP = 1024
CAP = LANES * SUBCAP
CHUNK = 128
NBUF = 8
LANES2 = 32
RPT2 = D // LANES2
DW = D // 2
ECHUNK = 16000
RPT = D // LANES


def _edge_partition_kernel():
    mesh = plsc.VectorSubcoreMesh(core_axis_name="c", subcore_axis_name="s")

    @functools.partial(
        pl.kernel,
        out_type=(
            jax.ShapeDtypeStruct((NW, CAP), jnp.int32),
            jax.ShapeDtypeStruct((NW, 128), jnp.int32),
        ),
        mesh=mesh,
        compiler_params=pltpu.CompilerParams(needs_layout_passes=False),
        scratch_types=[
            pltpu.VMEM((CAP,), jnp.int32),
            pltpu.VMEM((CAP,), jnp.int32),
            pltpu.VMEM((ECHUNK,), jnp.int32),
            pltpu.VMEM((ECHUNK,), jnp.int32),
            pltpu.VMEM((128,), jnp.int32),
        ],
    )
    def part(src_hbm, dst_hbm, lists_hbm, counts_hbm, lv, lvc, srcb, dstb,
             cv):
        wid = lax.axis_index("s") * 2 + lax.axis_index("c")
        lo = wid * NLOC
        pad = jnp.full((LANES,), TRASH, jnp.int32)

        def fill(i, _):
            lv[pl.ds(i * LANES, LANES)] = pad
            return 0

        lax.fori_loop(0, SUBCAP, fill, 0)

        lanebase = lax.iota(jnp.int32, LANES) * SUBCAP

        def outer(k, cnt):
            pltpu.sync_copy(src_hbm.at[pl.ds(k * ECHUNK, ECHUNK)], srcb)
            pltpu.sync_copy(dst_hbm.at[pl.ds(k * ECHUNK, ECHUNK)], dstb)

            def inner(j, cnt):
                sv = srcb[pl.ds(j * LANES, LANES)]
                dv = dstb[pl.ds(j * LANES, LANES)]
                dl = dv - lo
                m = (dl >= 0) & (dl < NLOC)
                pk = (sv << 9) | jnp.where(m, dl, 0)
                plsc.store_scatter(lv, [lanebase + cnt], pk, mask=m)
                return cnt + jnp.where(m, 1, 0).astype(jnp.int32)

            return lax.fori_loop(0, ECHUNK // LANES, inner, cnt)

        cnt = lax.fori_loop(0, N_EDGES // ECHUNK, outer,
                            jnp.zeros((LANES,), jnp.int32))

        base = jnp.int32(0)
        for l in range(LANES):
            nv = (cnt[l] + (LANES - 1)) >> 4
            src_base = l * SUBCAP

            def cp(j, _, b=base, s=src_base):
                lvc[pl.ds(b + j * LANES, LANES)] = lv[pl.ds(s + j * LANES,
                                                            LANES)]
                return 0

            lax.fori_loop(0, nv, cp, 0)
            base = base + nv * LANES
        for j in range(CHUNK // LANES):
            lvc[pl.ds(base + j * LANES, LANES)] = pad

        cv[pl.ds(0, LANES)] = jnp.full((LANES,), 1, jnp.int32) * base
        pltpu.sync_copy(lvc, lists_hbm.at[wid])
        pltpu.sync_copy(cv, counts_hbm.at[wid])

    return part


def _max_agg_kernel():
    mesh = plsc.VectorSubcoreMesh(core_axis_name="c", subcore_axis_name="s")

    @functools.partial(
        pl.kernel,
        out_type=jax.ShapeDtypeStruct((NPAD * DW,), jnp.int32),
        mesh=mesh,
        compiler_params=pltpu.CompilerParams(needs_layout_passes=False,
                                             use_tc_tiling_on_sc=False),
        scratch_types=[
            pltpu.VMEM((CAP,), jnp.int32),
            pltpu.VMEM((128,), jnp.int32),
            pltpu.VMEM((ACCR * DW,), jnp.int32),
        ] + [
            t for _ in range(NBUF) for t in (
                pltpu.VMEM((CHUNK,), jnp.int32),
                pltpu.VMEM((CHUNK,), jnp.int32),
                pltpu.VMEM((CHUNK, DW), jnp.int32),
                pltpu.SemaphoreType.DMA,
            )
        ],
    )
    def agg(h_hbm, lists_hbm, counts_hbm, out_hbm, lv, cv, acc, *bufargs):
        wid = lax.axis_index("s") * 2 + lax.axis_index("c")
        pltpu.sync_copy(lists_hbm.at[wid], lv)
        pltpu.sync_copy(counts_hbm.at[wid], cv)
        total = cv[pl.ds(0, LANES)][0]
        nch = (total + (CHUNK - 1)) // CHUNK

        ninf = jnp.full((LANES2,), -jnp.inf, jnp.bfloat16)
        zero = jnp.zeros((LANES2,), jnp.bfloat16)
        ninf32 = plsc.bitcast(ninf, jnp.int32)

        def initrow(rr, _):
            for k in range(RPT2):
                acc[pl.ds(rr * DW + k * LANES, LANES)] = ninf32
            return 0

        lax.fori_loop(0, ACCR, initrow, 0)

        bufs = tuple(tuple(bufargs[4 * b:4 * b + 4]) for b in range(NBUF))

        def issue(b, c):
            sidx, dloc, msg, sem = bufs[b]
            base = c * CHUNK

            def unpack(j, _):
                pk = lv[pl.ds(base + j * LANES, LANES)]
                sidx[pl.ds(j * LANES, LANES)] = pk >> 9
                dloc[pl.ds(j * LANES, LANES)] = pk & (2 ** 9 - 1)
                return 0

            lax.fori_loop(0, CHUNK // LANES, unpack, 0)
            pltpu.async_copy(h_hbm.at[sidx], msg, sem)

        for b in range(NBUF):
            @pl.when(nch > b)
            def _(b=b):
                issue(b, b)

        def louter(c2, _):
            for b in range(NBUF):
                c = c2 * NBUF + b
                sidx, dloc, msg, sem = bufs[b]

                @pl.when(c < nch)
                def _(c=c, sidx=sidx, dloc=dloc, msg=msg, sem=sem, b=b):
                    pltpu.make_async_copy(h_hbm.at[sidx], msg, sem).wait()

                    def do_group(g, _):
                        dv = dloc[pl.ds(g * LANES, LANES)]
                        for lane in range(LANES):
                            off = dv[lane] * DW
                            erow = g * LANES + lane
                            for r in range(RPT2):
                                a = plsc.bitcast(
                                    acc[pl.ds(off + r * LANES, LANES)],
                                    jnp.bfloat16)
                                mv = plsc.bitcast(
                                    msg[erow, pl.ds(r * LANES, LANES)],
                                    jnp.bfloat16)
                                acc[pl.ds(off + r * LANES, LANES)] = (
                                    plsc.bitcast(jnp.maximum(a, mv),
                                                 jnp.int32))
                        return 0

                    lax.fori_loop(0, CHUNK // LANES, do_group, 0)

                    @pl.when(c + NBUF < nch)
                    def _():
                        issue(b, c + NBUF)

            return 0

        lax.fori_loop(0, (nch + NBUF - 1) // NBUF, louter, 0)

        def fixrow(rr, _):
            for k in range(RPT2):
                v = plsc.bitcast(acc[pl.ds(rr * DW + k * LANES, LANES)],
                                 jnp.bfloat16)
                acc[pl.ds(rr * DW + k * LANES, LANES)] = (
                    plsc.bitcast(jnp.where(v == ninf, zero, v), jnp.int32))
            return 0

        lax.fori_loop(0, NLOC, fixrow, 0)
        pltpu.sync_copy(acc.at[pl.ds(0, NLOC * DW)],
                        out_hbm.at[pl.ds(wid * NLOC * DW, NLOC * DW)])

    return agg


_part = _edge_partition_kernel()
_agg = _max_agg_kernel()

_ROWS = NPAD // 4


def _tc_linear(h, w, b, silu_in):
    def body(h_ref, w_ref, b_ref, o_ref):
        hv = h_ref[...].astype(jnp.float32)
        if silu_in:
            hv = hv / (1.0 + jnp.exp(-hv))
        o_ref[...] = (lax.dot_general(
            hv, w_ref[...], (((1,), (1,)), ((), ())),
            preferred_element_type=jnp.float32,
            precision=lax.Precision.HIGHEST,
        ) + b_ref[...]).astype(jnp.bfloat16)

    return pl.pallas_call(
        body,
        grid=(NPAD // _ROWS,),
        in_specs=[
            pl.BlockSpec((_ROWS, D), lambda i: (i, 0)),
            pl.BlockSpec((D, D), lambda i: (0, 0)),
            pl.BlockSpec((1, D), lambda i: (0, 0)),
        ],
        out_specs=pl.BlockSpec((_ROWS, D), lambda i: (i, 0)),
        out_shape=jax.ShapeDtypeStruct((NPAD, D), jnp.bfloat16),
    )(h, w, b.reshape(1, D))


def kernel(x, edge_index, W0, b0, W1, b1, W2, b2, W3, b3, W4, b4):
    src = edge_index[0].astype(jnp.int32)
    dst = edge_index[1].astype(jnp.int32)
    xp = jnp.zeros((NPAD, D), jnp.float32).at[:N_NODES].set(x)

    lists, counts = _part(src, dst)

    ws = [W0, W1, W2, W3, W4]
    bs = [b0, b1, b2, b3, b4]
    h = xp
    for i in range(5):
        g = _tc_linear(h, ws[i], bs[i], silu_in=(i > 0))
        gp = lax.bitcast_convert_type(g.reshape(NPAD, DW, 2), jnp.int32)
        a = _agg(gp, lists, counts)
        h = lax.bitcast_convert_type(a.reshape(NPAD, DW),
                                     jnp.bfloat16).reshape(NPAD, D)
    return h[:N_NODES].astype(jnp.float32)
